# baseline (device time: 73613 ns/iter reference)
import jax
import jax.numpy as jnp
from jax import lax
from jax.experimental import pallas as pl
from jax.experimental.pallas import tpu as pltpu

N_DEV = 4


def kernel(x, w_mat):
    m_total, k_per = x.shape
    k_total, n = w_mat.shape
    m_per = m_total // N_DEV

    def body(x_hbm, w_hbm, out_hbm, acc_ref, xstage_ref, xb_ref, comm_ref,
             wstage_ref, xsems, send_sems, recv_sems, wsems, out_sem):
        my = lax.axis_index("i")

        def x_copy(row, slot):
            return pltpu.make_async_copy(
                x_hbm.at[pl.ds(row * m_per, m_per), :],
                xstage_ref.at[slot],
                xsems.at[slot],
            )

        xcs = []
        for d in range(1, N_DEV):
            c = x_copy(lax.rem(my + d, N_DEV), d - 1)
            c.start()
            xcs.append(c)
        xc_own = x_copy(my, N_DEV - 1)
        xc_own.start()

        def w_copy(kb, slot):
            return pltpu.make_async_copy(
                w_hbm.at[pl.ds(kb * k_per, k_per), :],
                wstage_ref.at[slot],
                wsems.at[slot],
            )

        wc = [w_copy(my, 0)]
        wc[0].start()

        barrier = pltpu.get_barrier_semaphore()
        for d in range(1, N_DEV):
            peer = lax.rem(my + d, N_DEV)
            pl.semaphore_signal(
                barrier, inc=1,
                device_id=(peer,), device_id_type=pl.DeviceIdType.MESH,
            )
        pl.semaphore_wait(barrier, N_DEV - 1)

        rdmas = []
        for d in range(1, N_DEV):
            peer = lax.rem(my + d, N_DEV)
            xcs[d - 1].wait()
            xb_ref[d - 1] = xstage_ref[d - 1].astype(jnp.bfloat16)
            rdma = pltpu.make_async_remote_copy(
                src_ref=xb_ref.at[d - 1],
                dst_ref=comm_ref.at[d - 1],
                send_sem=send_sems.at[d - 1],
                recv_sem=recv_sems.at[d - 1],
                device_id=(peer,),
                device_id_type=pl.DeviceIdType.MESH,
            )
            rdma.start()
            rdmas.append(rdma)

        wc.append(w_copy(lax.rem(my + N_DEV - 1, N_DEV), 1))
        wc[1].start()

        xc_own.wait()
        wc[0].wait()
        acc_ref[...] = jnp.dot(
            xstage_ref[N_DEV - 1].astype(jnp.bfloat16),
            wstage_ref[0].astype(jnp.bfloat16),
            preferred_element_type=jnp.float32,
        )

        for d in range(1, N_DEV):
            slot = d % 2
            if d + 1 < N_DEV:
                kb_next = lax.rem(my - (d + 1) + N_DEV, N_DEV)
                nxt = w_copy(kb_next, (d + 1) % 2)
                nxt.start()
                wc.append(nxt)
            rdmas[d - 1].wait_recv()
            wc[d].wait()
            acc_ref[...] += jnp.dot(
                comm_ref[d - 1],
                wstage_ref[slot].astype(jnp.bfloat16),
                preferred_element_type=jnp.float32,
            )

        for r in rdmas:
            r.wait_send()

        y = acc_ref[...]
        acc_ref[...] = y * jax.nn.sigmoid(y)
        out_copy = pltpu.make_async_copy(acc_ref, out_hbm, out_sem)
        out_copy.start()
        out_copy.wait()

    return pl.pallas_call(
        body,
        out_shape=jax.ShapeDtypeStruct((m_per, n), jnp.float32),
        in_specs=[
            pl.BlockSpec(memory_space=pl.ANY),
            pl.BlockSpec(memory_space=pl.ANY),
        ],
        out_specs=pl.BlockSpec(memory_space=pl.ANY),
        scratch_shapes=[
            pltpu.VMEM((m_per, n), jnp.float32),
            pltpu.VMEM((N_DEV, m_per, k_per), jnp.float32),
            pltpu.VMEM((N_DEV - 1, m_per, k_per), jnp.bfloat16),
            pltpu.VMEM((N_DEV - 1, m_per, k_per), jnp.bfloat16),
            pltpu.VMEM((2, k_per, n), jnp.float32),
            pltpu.SemaphoreType.DMA((N_DEV,)),
            pltpu.SemaphoreType.DMA((N_DEV - 1,)),
            pltpu.SemaphoreType.DMA((N_DEV - 1,)),
            pltpu.SemaphoreType.DMA((2,)),
            pltpu.SemaphoreType.DMA,
        ],
        compiler_params=pltpu.CompilerParams(
            collective_id=0,
            vmem_limit_bytes=100 * 1024 * 1024,
        ),
    )(x, w_mat)


# device time: 68645 ns/iter; 1.0724x vs baseline; 1.0724x over previous
import jax
import jax.numpy as jnp
from jax import lax
from jax.experimental import pallas as pl
from jax.experimental.pallas import tpu as pltpu

N_DEV = 4


def kernel(x, w_mat):
    m_total, k_per = x.shape
    k_total, n = w_mat.shape
    m_per = m_total // N_DEV

    def body(x_hbm, w_hbm, out_hbm, acc_ref, xstage_ref, xb_ref, comm_ref,
             wstage_ref, xsems, send_sems, recv_sems, wsems, out_sems):
        my = lax.axis_index("i")

        def x_copy(row, slot):
            return pltpu.make_async_copy(
                x_hbm.at[pl.ds(row * m_per, m_per), :],
                xstage_ref.at[slot],
                xsems.at[slot],
            )

        xcs = []
        for d in range(1, N_DEV):
            c = x_copy(lax.rem(my + d, N_DEV), d - 1)
            c.start()
            xcs.append(c)
        xc_own = x_copy(my, N_DEV - 1)
        xc_own.start()

        def w_copy(kb, slot):
            return pltpu.make_async_copy(
                w_hbm.at[pl.ds(kb * k_per, k_per), :],
                wstage_ref.at[slot],
                wsems.at[slot],
            )

        wc = [w_copy(my, 0)]
        wc[0].start()

        barrier = pltpu.get_barrier_semaphore()
        for d in range(1, N_DEV):
            peer = lax.rem(my + d, N_DEV)
            pl.semaphore_signal(
                barrier, inc=1,
                device_id=(peer,), device_id_type=pl.DeviceIdType.MESH,
            )
        pl.semaphore_wait(barrier, N_DEV - 1)

        rdmas = []
        for d in range(1, N_DEV):
            peer = lax.rem(my + d, N_DEV)
            xcs[d - 1].wait()
            xb_ref[d - 1] = xstage_ref[d - 1].astype(jnp.bfloat16)
            rdma = pltpu.make_async_remote_copy(
                src_ref=xb_ref.at[d - 1],
                dst_ref=comm_ref.at[d - 1],
                send_sem=send_sems.at[d - 1],
                recv_sem=recv_sems.at[d - 1],
                device_id=(peer,),
                device_id_type=pl.DeviceIdType.MESH,
            )
            rdma.start()
            rdmas.append(rdma)

        wc.append(w_copy(lax.rem(my + N_DEV - 1, N_DEV), 1))
        wc[1].start()

        xc_own.wait()
        wc[0].wait()
        acc_ref[...] = jnp.dot(
            xstage_ref[N_DEV - 1].astype(jnp.bfloat16),
            wstage_ref[0].astype(jnp.bfloat16),
            preferred_element_type=jnp.float32,
        )

        for d in range(1, N_DEV - 1):
            slot = d % 2
            kb_next = lax.rem(my - (d + 1) + N_DEV, N_DEV)
            nxt = w_copy(kb_next, (d + 1) % 2)
            nxt.start()
            wc.append(nxt)
            rdmas[d - 1].wait_recv()
            wc[d].wait()
            acc_ref[...] += jnp.dot(
                comm_ref[d - 1],
                wstage_ref[slot].astype(jnp.bfloat16),
                preferred_element_type=jnp.float32,
            )

        d = N_DEV - 1
        rdmas[d - 1].wait_recv()
        wc[d].wait()
        w_last = wstage_ref[d % 2].astype(jnp.bfloat16)
        n_chunks = 4
        m_c = m_per // n_chunks
        out_copies = []
        for c in range(n_chunks):
            rs = pl.ds(c * m_c, m_c)
            y = acc_ref[rs, :] + jnp.dot(
                comm_ref[d - 1][c * m_c:(c + 1) * m_c, :],
                w_last,
                preferred_element_type=jnp.float32,
            )
            acc_ref[rs, :] = y * jax.nn.sigmoid(y)
            oc = pltpu.make_async_copy(
                acc_ref.at[rs, :], out_hbm.at[rs, :], out_sems.at[c]
            )
            oc.start()
            out_copies.append(oc)

        for r in rdmas:
            r.wait_send()
        for oc in out_copies:
            oc.wait()

    return pl.pallas_call(
        body,
        out_shape=jax.ShapeDtypeStruct((m_per, n), jnp.float32),
        in_specs=[
            pl.BlockSpec(memory_space=pl.ANY),
            pl.BlockSpec(memory_space=pl.ANY),
        ],
        out_specs=pl.BlockSpec(memory_space=pl.ANY),
        scratch_shapes=[
            pltpu.VMEM((m_per, n), jnp.float32),
            pltpu.VMEM((N_DEV, m_per, k_per), jnp.float32),
            pltpu.VMEM((N_DEV - 1, m_per, k_per), jnp.bfloat16),
            pltpu.VMEM((N_DEV - 1, m_per, k_per), jnp.bfloat16),
            pltpu.VMEM((2, k_per, n), jnp.float32),
            pltpu.SemaphoreType.DMA((N_DEV,)),
            pltpu.SemaphoreType.DMA((N_DEV - 1,)),
            pltpu.SemaphoreType.DMA((N_DEV - 1,)),
            pltpu.SemaphoreType.DMA((2,)),
            pltpu.SemaphoreType.DMA((4,)),
        ],
        compiler_params=pltpu.CompilerParams(
            collective_id=0,
            vmem_limit_bytes=100 * 1024 * 1024,
        ),
    )(x, w_mat)


# device time: 68326 ns/iter; 1.0774x vs baseline; 1.0047x over previous
import jax
import jax.numpy as jnp
from jax import lax
from jax.experimental import pallas as pl
from jax.experimental.pallas import tpu as pltpu

N_DEV = 4


def kernel(x, w_mat):
    m_total, k_per = x.shape
    k_total, n = w_mat.shape
    m_per = m_total // N_DEV

    def body(x_hbm, w_hbm, out_hbm, acc_ref, xstage_ref, xb_ref, comm_ref,
             wstage_ref, xsems, send_sems, recv_sems, wsems, out_sems):
        my = lax.axis_index("i")

        m_h = m_per // 2

        def x_copy(row, slot, h):
            return pltpu.make_async_copy(
                x_hbm.at[pl.ds(row * m_per + h * m_h, m_h), :],
                xstage_ref.at[slot, pl.ds(h * m_h, m_h)],
                xsems.at[2 * slot + h],
            )

        xcs = []
        for d in range(1, N_DEV):
            row = lax.rem(my + d, N_DEV)
            halves = []
            for h in range(2):
                c = x_copy(row, d - 1, h)
                c.start()
                halves.append(c)
            xcs.append(halves)
        xc_own = []
        for h in range(2):
            c = x_copy(my, N_DEV - 1, h)
            c.start()
            xc_own.append(c)

        def w_copy(kb, slot):
            return pltpu.make_async_copy(
                w_hbm.at[pl.ds(kb * k_per, k_per), :],
                wstage_ref.at[slot],
                wsems.at[slot],
            )

        wc = [w_copy(my, 0)]
        wc[0].start()

        barrier = pltpu.get_barrier_semaphore()
        for d in range(1, N_DEV):
            peer = lax.rem(my + d, N_DEV)
            pl.semaphore_signal(
                barrier, inc=1,
                device_id=(peer,), device_id_type=pl.DeviceIdType.MESH,
            )
        pl.semaphore_wait(barrier, N_DEV - 1)

        rdmas = []
        for d in range(1, N_DEV):
            peer = lax.rem(my + d, N_DEV)
            halves = []
            for h in range(2):
                rs = pl.ds(h * m_h, m_h)
                xcs[d - 1][h].wait()
                xb_ref[d - 1, rs] = xstage_ref[d - 1][h * m_h:(h + 1) * m_h,
                                                      :].astype(jnp.bfloat16)
                rdma = pltpu.make_async_remote_copy(
                    src_ref=xb_ref.at[d - 1, rs],
                    dst_ref=comm_ref.at[d - 1, rs],
                    send_sem=send_sems.at[2 * (d - 1) + h],
                    recv_sem=recv_sems.at[2 * (d - 1) + h],
                    device_id=(peer,),
                    device_id_type=pl.DeviceIdType.MESH,
                )
                rdma.start()
                halves.append(rdma)
            rdmas.append(halves)

        wc.append(w_copy(lax.rem(my + N_DEV - 1, N_DEV), 1))
        wc[1].start()

        for c in xc_own:
            c.wait()
        wc[0].wait()
        acc_ref[...] = jnp.dot(
            xstage_ref[N_DEV - 1].astype(jnp.bfloat16),
            wstage_ref[0].astype(jnp.bfloat16),
            preferred_element_type=jnp.float32,
        )

        for d in range(1, N_DEV - 1):
            slot = d % 2
            kb_next = lax.rem(my - (d + 1) + N_DEV, N_DEV)
            nxt = w_copy(kb_next, (d + 1) % 2)
            nxt.start()
            wc.append(nxt)
            rdmas[d - 1][0].wait_recv()
            rdmas[d - 1][1].wait_recv()
            wc[d].wait()
            acc_ref[...] += jnp.dot(
                comm_ref[d - 1],
                wstage_ref[slot].astype(jnp.bfloat16),
                preferred_element_type=jnp.float32,
            )

        d = N_DEV - 1
        rdmas[d - 1][0].wait_recv()
        rdmas[d - 1][1].wait_recv()
        wc[d].wait()
        w_last = wstage_ref[d % 2].astype(jnp.bfloat16)
        n_chunks = 4
        m_c = m_per // n_chunks
        out_copies = []
        for c in range(n_chunks):
            rs = pl.ds(c * m_c, m_c)
            y = acc_ref[rs, :] + jnp.dot(
                comm_ref[d - 1][c * m_c:(c + 1) * m_c, :],
                w_last,
                preferred_element_type=jnp.float32,
            )
            acc_ref[rs, :] = y * jax.nn.sigmoid(y)
            oc = pltpu.make_async_copy(
                acc_ref.at[rs, :], out_hbm.at[rs, :], out_sems.at[c]
            )
            oc.start()
            out_copies.append(oc)

        for halves in rdmas:
            for r in halves:
                r.wait_send()
        for oc in out_copies:
            oc.wait()

    return pl.pallas_call(
        body,
        out_shape=jax.ShapeDtypeStruct((m_per, n), jnp.float32),
        in_specs=[
            pl.BlockSpec(memory_space=pl.ANY),
            pl.BlockSpec(memory_space=pl.ANY),
        ],
        out_specs=pl.BlockSpec(memory_space=pl.ANY),
        scratch_shapes=[
            pltpu.VMEM((m_per, n), jnp.float32),
            pltpu.VMEM((N_DEV, m_per, k_per), jnp.float32),
            pltpu.VMEM((N_DEV - 1, m_per, k_per), jnp.bfloat16),
            pltpu.VMEM((N_DEV - 1, m_per, k_per), jnp.bfloat16),
            pltpu.VMEM((2, k_per, n), jnp.float32),
            pltpu.SemaphoreType.DMA((2 * N_DEV,)),
            pltpu.SemaphoreType.DMA((2 * (N_DEV - 1),)),
            pltpu.SemaphoreType.DMA((2 * (N_DEV - 1),)),
            pltpu.SemaphoreType.DMA((2,)),
            pltpu.SemaphoreType.DMA((4,)),
        ],
        compiler_params=pltpu.CompilerParams(
            collective_id=0,
            vmem_limit_bytes=100 * 1024 * 1024,
        ),
    )(x, w_mat)
